# baseline (speedup 1.0000x reference)
import functools

import jax
import jax.numpy as jnp
from jax import lax
from jax.experimental import pallas as pl
from jax.experimental.pallas import tpu as pltpu
from jax.experimental.pallas import tpu_sc as plsc

IN_CH = 4096
N_ROWS = 8192
L = 16
NC, NS = 2, 16
NW = NC * NS
ROWS_PER_W = N_ROWS // NW
C = 8
CHUNKS = ROWS_PER_W // C
GROUPS = IN_CH // L

_mesh = plsc.VectorSubcoreMesh(
    core_axis_name="c", subcore_axis_name="s", num_cores=NC, num_subcores=NS)


@functools.partial(
    pl.kernel,
    out_type=jax.ShapeDtypeStruct((N_ROWS * IN_CH,), jnp.float32),
    mesh=_mesh,
    compiler_params=pltpu.CompilerParams(needs_layout_passes=False),
    scratch_types=[
        pltpu.VMEM((IN_CH,), jnp.int32),
        pltpu.VMEM((C * IN_CH,), jnp.float32),
        pltpu.VMEM((C * IN_CH,), jnp.float32),
    ],
)
def _permute(x_hbm, p_hbm, out_hbm, p_v, in_v, out_v):
    wid = lax.axis_index("s") * NC + lax.axis_index("c")
    row0 = wid * ROWS_PER_W
    pltpu.sync_copy(p_hbm, p_v)

    def chunk_body(ci, carry):
        base = pl.multiple_of((row0 + ci * C) * IN_CH, IN_CH)
        pltpu.sync_copy(x_hbm.at[pl.ds(base, C * IN_CH)], in_v)

        def g_body(g, carry2):
            col = pl.multiple_of(g * L, L)
            idx = p_v[pl.ds(col, L)]
            for r in range(C):
                val = plsc.load_gather(in_v, [idx + r * IN_CH])
                out_v[pl.ds(r * IN_CH + col, L)] = val
            return carry2

        lax.fori_loop(0, GROUPS, g_body, 0)
        pltpu.sync_copy(out_v, out_hbm.at[pl.ds(base, C * IN_CH)])
        return carry

    lax.fori_loop(0, CHUNKS, chunk_body, 0)


def kernel(x, p):
    out = _permute(x.reshape(-1), p.astype(jnp.int32))
    return (out.reshape(N_ROWS, IN_CH), 0)


# speedup vs baseline: 1.9663x; 1.9663x over previous
import functools

import jax
import jax.numpy as jnp
from jax import lax
from jax.experimental import pallas as pl
from jax.experimental.pallas import tpu as pltpu
from jax.experimental.pallas import tpu_sc as plsc

IN_CH = 4096
N_ROWS = 8192
L = 16
NC, NS = 2, 16
NW = NC * NS
ROWS_PER_W = N_ROWS // NW
C = 4
CHUNKS = ROWS_PER_W // C
NPAIR = CHUNKS // 2
GROUPS = IN_CH // L
CHUNK_W = C * IN_CH

_mesh = plsc.VectorSubcoreMesh(
    core_axis_name="c", subcore_axis_name="s", num_cores=NC, num_subcores=NS)


@functools.partial(
    pl.kernel,
    out_type=jax.ShapeDtypeStruct((N_ROWS * IN_CH,), jnp.float32),
    mesh=_mesh,
    compiler_params=pltpu.CompilerParams(needs_layout_passes=False),
    scratch_types=[
        pltpu.VMEM((IN_CH,), jnp.int32),
        pltpu.VMEM((CHUNK_W,), jnp.float32),
        pltpu.VMEM((CHUNK_W,), jnp.float32),
        pltpu.VMEM((CHUNK_W,), jnp.float32),
        pltpu.VMEM((CHUNK_W,), jnp.float32),
        pltpu.SemaphoreType.DMA,
        pltpu.SemaphoreType.DMA,
        pltpu.SemaphoreType.DMA,
        pltpu.SemaphoreType.DMA,
    ],
)
def _permute(x_hbm, p_hbm, out_hbm, p_v, in0, in1, out0, out1,
             sem_in0, sem_in1, sem_out0, sem_out1):
    wid = lax.axis_index("s") * NC + lax.axis_index("c")
    base0 = wid * ROWS_PER_W * IN_CH
    pltpu.sync_copy(p_hbm, p_v)

    def chunk_base(k):
        return pl.multiple_of(base0 + k * CHUNK_W, CHUNK_W)

    def start_in(k, buf, sem):
        pltpu.make_async_copy(
            x_hbm.at[pl.ds(chunk_base(k), CHUNK_W)], buf, sem).start()

    def start_out(k, buf, sem):
        pltpu.make_async_copy(
            buf, out_hbm.at[pl.ds(chunk_base(k), CHUNK_W)], sem).start()

    def permute_chunk(src, dst):
        @plsc.parallel_loop(0, GROUPS, unroll=4)
        def _(g):
            col = pl.multiple_of(g * L, L)
            idx = p_v[pl.ds(col, L)]
            for r in range(C):
                val = plsc.load_gather(src, [idx + r * IN_CH])
                dst[pl.ds(r * IN_CH + col, L)] = val

    start_in(0, in0, sem_in0)
    start_in(1, in1, sem_in1)

    def pair_body(i, carry):
        k = i * 2

        pltpu.make_async_copy(
            x_hbm.at[pl.ds(chunk_base(k), CHUNK_W)], in0, sem_in0).wait()

        @pl.when(i > 0)
        def _():
            pltpu.make_async_copy(
                out0, out_hbm.at[pl.ds(chunk_base(k - 2), CHUNK_W)],
                sem_out0).wait()

        permute_chunk(in0, out0)
        start_out(k, out0, sem_out0)

        @pl.when(i < NPAIR - 1)
        def _():
            start_in(k + 2, in0, sem_in0)

        pltpu.make_async_copy(
            x_hbm.at[pl.ds(chunk_base(k + 1), CHUNK_W)], in1, sem_in1).wait()

        @pl.when(i > 0)
        def _():
            pltpu.make_async_copy(
                out1, out_hbm.at[pl.ds(chunk_base(k - 1), CHUNK_W)],
                sem_out1).wait()

        permute_chunk(in1, out1)
        start_out(k + 1, out1, sem_out1)

        @pl.when(i < NPAIR - 1)
        def _():
            start_in(k + 3, in1, sem_in1)

        return carry

    lax.fori_loop(0, NPAIR, pair_body, 0)

    last = CHUNKS - 2
    pltpu.make_async_copy(
        out0, out_hbm.at[pl.ds(chunk_base(last), CHUNK_W)], sem_out0).wait()
    pltpu.make_async_copy(
        out1, out_hbm.at[pl.ds(chunk_base(last + 1), CHUNK_W)],
        sem_out1).wait()


def kernel(x, p):
    out = _permute(x.reshape(-1), p.astype(jnp.int32))
    return (out.reshape(N_ROWS, IN_CH), 0)
